# Initial kernel scaffold; baseline (speedup 1.0000x reference)
import jax
import jax.numpy as jnp
from jax.experimental import pallas as pl


def kernel(x, edge_index, params):
    raise NotImplementedError("write your pallas kernel here")


# speedup vs baseline: 6.7621x; 6.7621x over previous
import functools

import jax
import jax.numpy as jnp
from jax import lax
from jax.experimental import pallas as pl
from jax.experimental.pallas import tpu as pltpu
from jax.experimental.pallas import tpu_sc as plsc

N = 10000
NP = 10240
E = 320000
H = 128
NC = 2
NS = 16
NW = NC * NS
EPW = E // NW
C = 80
J = EPW // C
RPS = NP // NS

_sc_mesh = plsc.VectorSubcoreMesh(core_axis_name="c", subcore_axis_name="s")


@functools.partial(
    pl.kernel,
    out_type=jax.ShapeDtypeStruct((NC * NP, H), jnp.float32),
    mesh=_sc_mesh,
    scratch_types=[
        pltpu.VMEM((J, C), jnp.int32),
        pltpu.VMEM((J, C), jnp.int32),
        pltpu.VMEM((C, H), jnp.float32),
        pltpu.VMEM_SHARED((NP, H), jnp.float32),
        pltpu.SemaphoreType.DMA,
    ],
)
def _sc_agg(h_hbm, srcw, dstw, zeros_hbm, out_hbm, src_v, dst_v, gbuf, acc, sem):
    c = lax.axis_index("c")
    s = lax.axis_index("s")
    wid = c * NS + s
    pltpu.sync_copy(srcw.at[wid], src_v)
    pltpu.sync_copy(dstw.at[wid], dst_v)
    row0 = s * RPS
    pltpu.sync_copy(zeros_hbm.at[pl.ds(row0, RPS)], acc.at[pl.ds(row0, RPS)])
    plsc.subcore_barrier()

    def body(j, carry):
        pltpu.async_copy(h_hbm.at[src_v.at[j]], gbuf, sem).wait()
        pltpu.sync_copy(gbuf, acc.at[dst_v.at[j]], add=True)
        return carry

    lax.fori_loop(0, J, body, 0)
    plsc.subcore_barrier()
    pltpu.sync_copy(acc.at[pl.ds(row0, RPS)],
                    out_hbm.at[pl.ds(c * NP + row0, RPS)])


@functools.partial(
    pl.kernel,
    out_type=jax.ShapeDtypeStruct((NC * NP, H), jnp.float32),
    mesh=_sc_mesh,
    scratch_types=[
        pltpu.VMEM((J, C), jnp.int32),
        pltpu.VMEM((C, H), jnp.float32),
        pltpu.VMEM_SHARED((NP, H), jnp.float32),
    ],
)
def _sc_deg(dstw, zeros_hbm, ones_hbm, out_hbm, dst_v, obuf, acc):
    c = lax.axis_index("c")
    s = lax.axis_index("s")
    wid = c * NS + s
    pltpu.sync_copy(dstw.at[wid], dst_v)
    pltpu.sync_copy(ones_hbm, obuf)
    row0 = s * RPS
    pltpu.sync_copy(zeros_hbm.at[pl.ds(row0, RPS)], acc.at[pl.ds(row0, RPS)])
    plsc.subcore_barrier()

    def body(j, carry):
        pltpu.sync_copy(obuf, acc.at[dst_v.at[j]], add=True)
        return carry

    lax.fori_loop(0, J, body, 0)
    plsc.subcore_barrier()
    pltpu.sync_copy(acc.at[pl.ds(row0, RPS)],
                    out_hbm.at[pl.ds(c * NP + row0, RPS)])


BT = 2048
GT = NP // BT


def _enc_body(x_ref, w1_ref, b1_ref, w2_ref, b2_ref, h_ref, r2_ref):
    i = pl.program_id(0)
    x = x_ref[...]
    r = jnp.maximum(
        jnp.dot(x, w1_ref[...], preferred_element_type=jnp.float32) + b1_ref[...],
        0.0)
    h_ref[...] = (
        jnp.dot(r, w2_ref[...], preferred_element_type=jnp.float32) + b2_ref[...])
    col = lax.broadcasted_iota(jnp.int32, (BT, H), 1)
    m2 = jnp.max(jnp.sum(jnp.where(col < 3, x * x, 0.0), axis=1)).reshape(1, 1)

    @pl.when(i == 0)
    def _():
        r2_ref[...] = m2

    @pl.when(i != 0)
    def _():
        r2_ref[...] = jnp.maximum(r2_ref[...], m2)


_W = pl.BlockSpec((H, H), lambda i: (0, 0))
_B = pl.BlockSpec((1, H), lambda i: (0, 0))
_ROWS = pl.BlockSpec((BT, H), lambda i: (i, 0))

_enc = pl.pallas_call(
    _enc_body,
    grid=(GT,),
    in_specs=[_ROWS, _W, _B, _W, _B],
    out_specs=[_ROWS, pl.BlockSpec((1, 1), lambda i: (0, 0))],
    out_shape=[jax.ShapeDtypeStruct((NP, H), jnp.float32),
               jax.ShapeDtypeStruct((1, 1), jnp.float32)],
)


def _blk_body(p0_ref, p1_ref, d0_ref, d1_ref, h_ref, wl_ref, bl_ref, wr_ref,
              g_ref, be_ref, o_ref):
    deg = d0_ref[:, 0:1] + d1_ref[:, 0:1]
    agg = (p0_ref[...] + p1_ref[...]) / jnp.maximum(deg, 1.0)
    h = h_ref[...]
    z = (jnp.dot(agg, wl_ref[...], preferred_element_type=jnp.float32)
         + bl_ref[...]
         + jnp.dot(h, wr_ref[...], preferred_element_type=jnp.float32))
    mu = jnp.mean(z, axis=1, keepdims=True)
    zc = z - mu
    var = jnp.mean(zc * zc, axis=1, keepdims=True)
    ln = zc / jnp.sqrt(var + 1e-5) * g_ref[...] + be_ref[...]
    o_ref[...] = h + jnp.maximum(ln, 0.0)


_blk = pl.pallas_call(
    _blk_body,
    grid=(GT,),
    in_specs=[_ROWS,
              pl.BlockSpec((BT, H), lambda i: (i + GT, 0)),
              _ROWS,
              pl.BlockSpec((BT, H), lambda i: (i + GT, 0)),
              _ROWS, _W, _B, _W, _B, _B],
    out_specs=_ROWS,
    out_shape=jax.ShapeDtypeStruct((NP, H), jnp.float32),
)


def _heads_body(h_ref, r2_ref, dw1, db1, dw2, db2, dw3, db3,
                sw1, sb1, sw2, sb2, sw3, sb3, o_ref):
    h = h_ref[...]
    ratio = jnp.sqrt(r2_ref[...])
    d = jnp.maximum(
        jnp.dot(h, dw1[...], preferred_element_type=jnp.float32) + db1[...], 0.0)
    d = jnp.maximum(
        jnp.dot(d, dw2[...], preferred_element_type=jnp.float32) + db2[...], 0.0)
    d = jnp.dot(d, dw3[...], preferred_element_type=jnp.float32) + db3[...]
    s = jnp.maximum(
        jnp.dot(h, sw1[...], preferred_element_type=jnp.float32) + sb1[...], 0.0)
    s = jnp.maximum(
        jnp.dot(s, sw2[...], preferred_element_type=jnp.float32) + sb2[...], 0.0)
    s = jnp.dot(s, sw3[...], preferred_element_type=jnp.float32) + sb3[...]
    o_ref[...] = jnp.concatenate([d, s], axis=1) * ratio


_H2 = H // 2
_heads = pl.pallas_call(
    _heads_body,
    grid=(GT,),
    in_specs=[_ROWS,
              pl.BlockSpec((1, 1), lambda i: (0, 0)),
              _W, _B,
              pl.BlockSpec((H, _H2), lambda i: (0, 0)),
              pl.BlockSpec((1, _H2), lambda i: (0, 0)),
              pl.BlockSpec((_H2, 3), lambda i: (0, 0)),
              pl.BlockSpec((1, 3), lambda i: (0, 0)),
              _W, _B,
              pl.BlockSpec((H, _H2), lambda i: (0, 0)),
              pl.BlockSpec((1, _H2), lambda i: (0, 0)),
              pl.BlockSpec((_H2, 1), lambda i: (0, 0)),
              pl.BlockSpec((1, 1), lambda i: (0, 0))],
    out_specs=pl.BlockSpec((BT, 4), lambda i: (i, 0)),
    out_shape=jax.ShapeDtypeStruct((NP, 4), jnp.float32),
)


def kernel(x, edge_index, params):
    src = edge_index[0].astype(jnp.int32).reshape(NW, J, C)
    dst = edge_index[1].astype(jnp.int32).reshape(NW, J, C)
    zeros_nh = jnp.zeros((NP, H), jnp.float32)
    ones_ch = jnp.ones((C, H), jnp.float32)

    degp = _sc_deg(dst, zeros_nh, ones_ch)

    b = lambda v: v.reshape(1, -1)
    x_p = jnp.pad(x, ((0, NP - N), (0, 0)))
    h, r2 = _enc(x_p, params['enc_W1'], b(params['enc_b1']),
                 params['enc_W2'], b(params['enc_b2']))

    for blk in params['blocks']:
        parts = _sc_agg(h, src, dst, zeros_nh)
        h = _blk(parts, parts, degp, degp, h,
                 blk['Wl'], b(blk['bl']), blk['Wr'],
                 b(blk['gamma']), b(blk['beta']))

    out = _heads(h, r2,
                  params['d_W1'], b(params['d_b1']),
                  params['d_W2'], b(params['d_b2']),
                  params['d_W3'], b(params['d_b3']),
                  params['s_W1'], b(params['s_b1']),
                  params['s_W2'], b(params['s_b2']),
                  params['s_W3'], b(params['s_b3']))
    return out[:N]


# speedup vs baseline: 9.5362x; 1.4103x over previous
import functools

import jax
import jax.numpy as jnp
from jax import lax
from jax.experimental import pallas as pl
from jax.experimental.pallas import tpu as pltpu
from jax.experimental.pallas import tpu_sc as plsc

N = 10000
NP = 10240
E = 320000
H = 128
NC = 2
NS = 16
NW = NC * NS
EPW = E // NW
C = 80
J = EPW // C
G = 5
JG = J // G
RPS = NP // NS

_sc_mesh = plsc.VectorSubcoreMesh(core_axis_name="c", subcore_axis_name="s")


@functools.partial(
    pl.kernel,
    out_type=jax.ShapeDtypeStruct((NC * NP, H), jnp.float32),
    mesh=_sc_mesh,
    scratch_types=[
        pltpu.VMEM((JG, C), jnp.int32),
        pltpu.VMEM((JG, C), jnp.int32),
        pltpu.VMEM((C, H), jnp.float32),
        pltpu.VMEM((C, H), jnp.float32),
        pltpu.VMEM_SHARED((NP, H), jnp.float32),
        pltpu.SemaphoreType.DMA,
        pltpu.SemaphoreType.DMA,
    ],
)
def _sc_agg(h_hbm, srcw, dstw, zeros_hbm, out_hbm, src_v, dst_v, g0, g1, acc,
            sem0, sem1):
    c = lax.axis_index("c")
    s = lax.axis_index("s")
    wid = c * NS + s
    row0 = s * RPS
    pltpu.sync_copy(zeros_hbm.at[pl.ds(row0, RPS)], acc.at[pl.ds(row0, RPS)])
    plsc.subcore_barrier()

    def group(g, carry):
        pltpu.sync_copy(srcw.at[g, wid], src_v)
        pltpu.sync_copy(dstw.at[g, wid], dst_v)
        pltpu.async_copy(h_hbm.at[src_v.at[0]], g0, sem0)

        def body(jj, carry2):
            j0 = 2 * jj
            pltpu.async_copy(h_hbm.at[src_v.at[j0 + 1]], g1, sem1)
            pltpu.make_async_copy(h_hbm.at[src_v.at[j0]], g0, sem0).wait()
            pltpu.sync_copy(g0, acc.at[dst_v.at[j0]], add=True)
            pltpu.async_copy(h_hbm.at[src_v.at[j0 + 2]], g0, sem0)
            pltpu.make_async_copy(h_hbm.at[src_v.at[j0 + 1]], g1, sem1).wait()
            pltpu.sync_copy(g1, acc.at[dst_v.at[j0 + 1]], add=True)
            return carry2

        lax.fori_loop(0, (JG - 1) // 2, body, 0)
        pltpu.make_async_copy(h_hbm.at[src_v.at[JG - 1]], g0, sem0).wait()
        pltpu.sync_copy(g0, acc.at[dst_v.at[JG - 1]], add=True)
        return carry

    lax.fori_loop(0, G, group, 0)
    plsc.subcore_barrier()
    pltpu.sync_copy(acc.at[pl.ds(row0, RPS)],
                    out_hbm.at[pl.ds(c * NP + row0, RPS)])


@functools.partial(
    pl.kernel,
    out_type=jax.ShapeDtypeStruct((NC * NP, H), jnp.float32),
    mesh=_sc_mesh,
    scratch_types=[
        pltpu.VMEM((J, C), jnp.int32),
        pltpu.VMEM((C, H), jnp.float32),
        pltpu.VMEM_SHARED((NP, H), jnp.float32),
    ],
)
def _sc_deg(dstw, zeros_hbm, ones_hbm, out_hbm, dst_v, obuf, acc):
    c = lax.axis_index("c")
    s = lax.axis_index("s")
    wid = c * NS + s
    pltpu.sync_copy(dstw.at[wid], dst_v)
    pltpu.sync_copy(ones_hbm, obuf)
    row0 = s * RPS
    pltpu.sync_copy(zeros_hbm.at[pl.ds(row0, RPS)], acc.at[pl.ds(row0, RPS)])
    plsc.subcore_barrier()

    def body(j, carry):
        pltpu.sync_copy(obuf, acc.at[dst_v.at[j]], add=True)
        return carry

    lax.fori_loop(0, J, body, 0)
    plsc.subcore_barrier()
    pltpu.sync_copy(acc.at[pl.ds(row0, RPS)],
                    out_hbm.at[pl.ds(c * NP + row0, RPS)])


BT = 2048
GT = NP // BT


def _enc_body(x_ref, w1_ref, b1_ref, w2_ref, b2_ref, h_ref, r2_ref):
    i = pl.program_id(0)
    x = x_ref[...]
    r = jnp.maximum(
        jnp.dot(x, w1_ref[...], preferred_element_type=jnp.float32) + b1_ref[...],
        0.0)
    h_ref[...] = (
        jnp.dot(r, w2_ref[...], preferred_element_type=jnp.float32) + b2_ref[...])
    col = lax.broadcasted_iota(jnp.int32, (BT, H), 1)
    m2 = jnp.max(jnp.sum(jnp.where(col < 3, x * x, 0.0), axis=1)).reshape(1, 1)

    @pl.when(i == 0)
    def _():
        r2_ref[...] = m2

    @pl.when(i != 0)
    def _():
        r2_ref[...] = jnp.maximum(r2_ref[...], m2)


_W = pl.BlockSpec((H, H), lambda i: (0, 0))
_B = pl.BlockSpec((1, H), lambda i: (0, 0))
_ROWS = pl.BlockSpec((BT, H), lambda i: (i, 0))

_enc = pl.pallas_call(
    _enc_body,
    grid=(GT,),
    in_specs=[_ROWS, _W, _B, _W, _B],
    out_specs=[_ROWS, pl.BlockSpec((1, 1), lambda i: (0, 0))],
    out_shape=[jax.ShapeDtypeStruct((NP, H), jnp.float32),
               jax.ShapeDtypeStruct((1, 1), jnp.float32)],
)


def _blk_body(p0_ref, p1_ref, d0_ref, d1_ref, h_ref, wl_ref, bl_ref, wr_ref,
              g_ref, be_ref, o_ref):
    deg = d0_ref[:, 0:1] + d1_ref[:, 0:1]
    agg = (p0_ref[...] + p1_ref[...]) / jnp.maximum(deg, 1.0)
    h = h_ref[...]
    z = (jnp.dot(agg, wl_ref[...], preferred_element_type=jnp.float32)
         + bl_ref[...]
         + jnp.dot(h, wr_ref[...], preferred_element_type=jnp.float32))
    mu = jnp.mean(z, axis=1, keepdims=True)
    zc = z - mu
    var = jnp.mean(zc * zc, axis=1, keepdims=True)
    ln = zc / jnp.sqrt(var + 1e-5) * g_ref[...] + be_ref[...]
    o_ref[...] = h + jnp.maximum(ln, 0.0)


_blk = pl.pallas_call(
    _blk_body,
    grid=(GT,),
    in_specs=[_ROWS,
              pl.BlockSpec((BT, H), lambda i: (i + GT, 0)),
              _ROWS,
              pl.BlockSpec((BT, H), lambda i: (i + GT, 0)),
              _ROWS, _W, _B, _W, _B, _B],
    out_specs=_ROWS,
    out_shape=jax.ShapeDtypeStruct((NP, H), jnp.float32),
)


def _heads_body(h_ref, r2_ref, dw1, db1, dw2, db2, dw3, db3,
                sw1, sb1, sw2, sb2, sw3, sb3, o_ref):
    h = h_ref[...]
    ratio = jnp.sqrt(r2_ref[...])
    d = jnp.maximum(
        jnp.dot(h, dw1[...], preferred_element_type=jnp.float32) + db1[...], 0.0)
    d = jnp.maximum(
        jnp.dot(d, dw2[...], preferred_element_type=jnp.float32) + db2[...], 0.0)
    d = jnp.dot(d, dw3[...], preferred_element_type=jnp.float32) + db3[...]
    s = jnp.maximum(
        jnp.dot(h, sw1[...], preferred_element_type=jnp.float32) + sb1[...], 0.0)
    s = jnp.maximum(
        jnp.dot(s, sw2[...], preferred_element_type=jnp.float32) + sb2[...], 0.0)
    s = jnp.dot(s, sw3[...], preferred_element_type=jnp.float32) + sb3[...]
    o_ref[...] = jnp.concatenate([d, s], axis=1) * ratio


_H2 = H // 2
_heads = pl.pallas_call(
    _heads_body,
    grid=(GT,),
    in_specs=[_ROWS,
              pl.BlockSpec((1, 1), lambda i: (0, 0)),
              _W, _B,
              pl.BlockSpec((H, _H2), lambda i: (0, 0)),
              pl.BlockSpec((1, _H2), lambda i: (0, 0)),
              pl.BlockSpec((_H2, 3), lambda i: (0, 0)),
              pl.BlockSpec((1, 3), lambda i: (0, 0)),
              _W, _B,
              pl.BlockSpec((H, _H2), lambda i: (0, 0)),
              pl.BlockSpec((1, _H2), lambda i: (0, 0)),
              pl.BlockSpec((_H2, 1), lambda i: (0, 0)),
              pl.BlockSpec((1, 1), lambda i: (0, 0))],
    out_specs=pl.BlockSpec((BT, 4), lambda i: (i, 0)),
    out_shape=jax.ShapeDtypeStruct((NP, 4), jnp.float32),
)


def kernel(x, edge_index, params):
    src = edge_index[0].astype(jnp.int32).reshape(NW, J, C)
    dst = edge_index[1].astype(jnp.int32).reshape(NW, J, C)
    src_g = src.reshape(NW, G, JG, C).transpose(1, 0, 2, 3)
    dst_g = dst.reshape(NW, G, JG, C).transpose(1, 0, 2, 3)
    zeros_nh = jnp.zeros((NP, H), jnp.float32)
    ones_ch = jnp.ones((C, H), jnp.float32)

    degp = _sc_deg(dst, zeros_nh, ones_ch)

    b = lambda v: v.reshape(1, -1)
    x_p = jnp.pad(x, ((0, NP - N), (0, 0)))
    h, r2 = _enc(x_p, params['enc_W1'], b(params['enc_b1']),
                 params['enc_W2'], b(params['enc_b2']))

    for blk in params['blocks']:
        parts = _sc_agg(h, src_g, dst_g, zeros_nh)
        h = _blk(parts, parts, degp, degp, h,
                 blk['Wl'], b(blk['bl']), blk['Wr'],
                 b(blk['gamma']), b(blk['beta']))

    out = _heads(h, r2,
                  params['d_W1'], b(params['d_b1']),
                  params['d_W2'], b(params['d_b2']),
                  params['d_W3'], b(params['d_b3']),
                  params['s_W1'], b(params['s_b1']),
                  params['s_W2'], b(params['s_b2']),
                  params['s_W3'], b(params['s_b3']))
    return out[:N]


# speedup vs baseline: 10.6128x; 1.1129x over previous
import functools

import jax
import jax.numpy as jnp
from jax import lax
from jax.experimental import pallas as pl
from jax.experimental.pallas import tpu as pltpu
from jax.experimental.pallas import tpu_sc as plsc

N = 10000
NP = 10240
E = 320000
H = 128
NC = 2
NS = 16
NW = NC * NS
EPW = E // NW
C = 80
J = EPW // C
G = 5
JG = J // G
RPS = NP // NS

_sc_mesh = plsc.VectorSubcoreMesh(core_axis_name="c", subcore_axis_name="s")


@functools.partial(
    pl.kernel,
    out_type=jax.ShapeDtypeStruct((NC * NP, H), jnp.float32),
    mesh=_sc_mesh,
    scratch_types=[
        pltpu.VMEM((JG, C), jnp.int32),
        pltpu.VMEM((JG, C), jnp.int32),
        pltpu.VMEM((C, H), jnp.float32),
        pltpu.VMEM((C, H), jnp.float32),
        pltpu.VMEM((C, H), jnp.float32),
        pltpu.VMEM_SHARED((NP, H), jnp.float32),
        pltpu.SemaphoreType.DMA,
        pltpu.SemaphoreType.DMA,
        pltpu.SemaphoreType.DMA,
    ],
)
def _sc_agg(h_hbm, srcw, dstw, zeros_hbm, out_hbm, src_v, dst_v, g0, g1, g2,
            acc, sem0, sem1, sem2):
    c = lax.axis_index("c")
    s = lax.axis_index("s")
    wid = c * NS + s
    row0 = s * RPS
    pltpu.sync_copy(zeros_hbm.at[pl.ds(row0, RPS)], acc.at[pl.ds(row0, RPS)])
    plsc.subcore_barrier()

    bufs = (g0, g1, g2)
    sems = (sem0, sem1, sem2)
    ND = 3
    T = (JG - ND) // ND

    def group(g, carry):
        pltpu.sync_copy(srcw.at[g, wid], src_v)
        pltpu.sync_copy(dstw.at[g, wid], dst_v)
        for k in range(ND):
            pltpu.async_copy(h_hbm.at[src_v.at[k]], bufs[k], sems[k])

        def body(t, carry2):
            j0 = ND * t
            for k in range(ND):
                j = j0 + k
                pltpu.make_async_copy(h_hbm.at[src_v.at[j]], bufs[k], sems[k]).wait()
                pltpu.sync_copy(bufs[k], acc.at[dst_v.at[j]], add=True)
                pltpu.async_copy(h_hbm.at[src_v.at[j + ND]], bufs[k], sems[k])
            return carry2

        lax.fori_loop(0, T, body, 0)
        for j in range(ND * T, JG):
            k = j % ND
            pltpu.make_async_copy(h_hbm.at[src_v.at[j]], bufs[k], sems[k]).wait()
            pltpu.sync_copy(bufs[k], acc.at[dst_v.at[j]], add=True)
            if j + ND < JG:
                pltpu.async_copy(h_hbm.at[src_v.at[j + ND]], bufs[k], sems[k])
        return carry

    lax.fori_loop(0, G, group, 0)
    plsc.subcore_barrier()
    pltpu.sync_copy(acc.at[pl.ds(row0, RPS)],
                    out_hbm.at[pl.ds(c * NP + row0, RPS)])


@functools.partial(
    pl.kernel,
    out_type=jax.ShapeDtypeStruct((NC * NP, H), jnp.float32),
    mesh=_sc_mesh,
    scratch_types=[
        pltpu.VMEM((J, C), jnp.int32),
        pltpu.VMEM((C, H), jnp.float32),
        pltpu.VMEM_SHARED((NP, H), jnp.float32),
    ],
)
def _sc_deg(dstw, zeros_hbm, ones_hbm, out_hbm, dst_v, obuf, acc):
    c = lax.axis_index("c")
    s = lax.axis_index("s")
    wid = c * NS + s
    pltpu.sync_copy(dstw.at[wid], dst_v)
    pltpu.sync_copy(ones_hbm, obuf)
    row0 = s * RPS
    pltpu.sync_copy(zeros_hbm.at[pl.ds(row0, RPS)], acc.at[pl.ds(row0, RPS)])
    plsc.subcore_barrier()

    def body(j, carry):
        pltpu.sync_copy(obuf, acc.at[dst_v.at[j]], add=True)
        return carry

    lax.fori_loop(0, J, body, 0)
    plsc.subcore_barrier()
    pltpu.sync_copy(acc.at[pl.ds(row0, RPS)],
                    out_hbm.at[pl.ds(c * NP + row0, RPS)])


BT = 2048
GT = NP // BT


def _enc_body(x_ref, w1_ref, b1_ref, w2_ref, b2_ref, h_ref, r2_ref):
    i = pl.program_id(0)
    x = x_ref[...]
    r = jnp.maximum(
        jnp.dot(x, w1_ref[...], preferred_element_type=jnp.float32) + b1_ref[...],
        0.0)
    h_ref[...] = (
        jnp.dot(r, w2_ref[...], preferred_element_type=jnp.float32) + b2_ref[...])
    col = lax.broadcasted_iota(jnp.int32, (BT, H), 1)
    m2 = jnp.max(jnp.sum(jnp.where(col < 3, x * x, 0.0), axis=1)).reshape(1, 1)

    @pl.when(i == 0)
    def _():
        r2_ref[...] = m2

    @pl.when(i != 0)
    def _():
        r2_ref[...] = jnp.maximum(r2_ref[...], m2)


_W = pl.BlockSpec((H, H), lambda i: (0, 0))
_B = pl.BlockSpec((1, H), lambda i: (0, 0))
_ROWS = pl.BlockSpec((BT, H), lambda i: (i, 0))

_enc = pl.pallas_call(
    _enc_body,
    grid=(GT,),
    in_specs=[_ROWS, _W, _B, _W, _B],
    out_specs=[_ROWS, pl.BlockSpec((1, 1), lambda i: (0, 0))],
    out_shape=[jax.ShapeDtypeStruct((NP, H), jnp.float32),
               jax.ShapeDtypeStruct((1, 1), jnp.float32)],
)


def _blk_body(p0_ref, p1_ref, d0_ref, d1_ref, h_ref, wl_ref, bl_ref, wr_ref,
              g_ref, be_ref, o_ref):
    deg = d0_ref[:, 0:1] + d1_ref[:, 0:1]
    agg = (p0_ref[...] + p1_ref[...]) / jnp.maximum(deg, 1.0)
    h = h_ref[...]
    z = (jnp.dot(agg, wl_ref[...], preferred_element_type=jnp.float32)
         + bl_ref[...]
         + jnp.dot(h, wr_ref[...], preferred_element_type=jnp.float32))
    mu = jnp.mean(z, axis=1, keepdims=True)
    zc = z - mu
    var = jnp.mean(zc * zc, axis=1, keepdims=True)
    ln = zc / jnp.sqrt(var + 1e-5) * g_ref[...] + be_ref[...]
    o_ref[...] = h + jnp.maximum(ln, 0.0)


_blk = pl.pallas_call(
    _blk_body,
    grid=(GT,),
    in_specs=[_ROWS,
              pl.BlockSpec((BT, H), lambda i: (i + GT, 0)),
              _ROWS,
              pl.BlockSpec((BT, H), lambda i: (i + GT, 0)),
              _ROWS, _W, _B, _W, _B, _B],
    out_specs=_ROWS,
    out_shape=jax.ShapeDtypeStruct((NP, H), jnp.float32),
)


def _heads_body(h_ref, r2_ref, dw1, db1, dw2, db2, dw3, db3,
                sw1, sb1, sw2, sb2, sw3, sb3, o_ref):
    h = h_ref[...]
    ratio = jnp.sqrt(r2_ref[...])
    d = jnp.maximum(
        jnp.dot(h, dw1[...], preferred_element_type=jnp.float32) + db1[...], 0.0)
    d = jnp.maximum(
        jnp.dot(d, dw2[...], preferred_element_type=jnp.float32) + db2[...], 0.0)
    d = jnp.dot(d, dw3[...], preferred_element_type=jnp.float32) + db3[...]
    s = jnp.maximum(
        jnp.dot(h, sw1[...], preferred_element_type=jnp.float32) + sb1[...], 0.0)
    s = jnp.maximum(
        jnp.dot(s, sw2[...], preferred_element_type=jnp.float32) + sb2[...], 0.0)
    s = jnp.dot(s, sw3[...], preferred_element_type=jnp.float32) + sb3[...]
    o_ref[...] = jnp.concatenate([d, s], axis=1) * ratio


_H2 = H // 2
_heads = pl.pallas_call(
    _heads_body,
    grid=(GT,),
    in_specs=[_ROWS,
              pl.BlockSpec((1, 1), lambda i: (0, 0)),
              _W, _B,
              pl.BlockSpec((H, _H2), lambda i: (0, 0)),
              pl.BlockSpec((1, _H2), lambda i: (0, 0)),
              pl.BlockSpec((_H2, 3), lambda i: (0, 0)),
              pl.BlockSpec((1, 3), lambda i: (0, 0)),
              _W, _B,
              pl.BlockSpec((H, _H2), lambda i: (0, 0)),
              pl.BlockSpec((1, _H2), lambda i: (0, 0)),
              pl.BlockSpec((_H2, 1), lambda i: (0, 0)),
              pl.BlockSpec((1, 1), lambda i: (0, 0))],
    out_specs=pl.BlockSpec((BT, 4), lambda i: (i, 0)),
    out_shape=jax.ShapeDtypeStruct((NP, 4), jnp.float32),
)


def kernel(x, edge_index, params):
    src = edge_index[0].astype(jnp.int32).reshape(NW, J, C)
    dst = edge_index[1].astype(jnp.int32).reshape(NW, J, C)
    src_g = src.reshape(NW, G, JG, C).transpose(1, 0, 2, 3)
    dst_g = dst.reshape(NW, G, JG, C).transpose(1, 0, 2, 3)
    zeros_nh = jnp.zeros((NP, H), jnp.float32)
    ones_ch = jnp.ones((C, H), jnp.float32)

    degp = _sc_deg(dst, zeros_nh, ones_ch)

    b = lambda v: v.reshape(1, -1)
    x_p = jnp.pad(x, ((0, NP - N), (0, 0)))
    h, r2 = _enc(x_p, params['enc_W1'], b(params['enc_b1']),
                 params['enc_W2'], b(params['enc_b2']))

    for blk in params['blocks']:
        parts = _sc_agg(h, src_g, dst_g, zeros_nh)
        h = _blk(parts, parts, degp, degp, h,
                 blk['Wl'], b(blk['bl']), blk['Wr'],
                 b(blk['gamma']), b(blk['beta']))

    out = _heads(h, r2,
                  params['d_W1'], b(params['d_b1']),
                  params['d_W2'], b(params['d_b2']),
                  params['d_W3'], b(params['d_b3']),
                  params['s_W1'], b(params['s_b1']),
                  params['s_W2'], b(params['s_b2']),
                  params['s_W3'], b(params['s_b3']))
    return out[:N]


# speedup vs baseline: 10.8098x; 1.0186x over previous
import functools

import jax
import jax.numpy as jnp
from jax import lax
from jax.experimental import pallas as pl
from jax.experimental.pallas import tpu as pltpu
from jax.experimental.pallas import tpu_sc as plsc

N = 10000
NP = 10240
E = 320000
H = 128
NC = 2
NS = 16
NW = NC * NS
EPW = E // NW
C = 80
J = EPW // C
G = 5
JG = J // G
RPS = NP // NS

_sc_mesh = plsc.VectorSubcoreMesh(core_axis_name="c", subcore_axis_name="s")


@functools.partial(
    pl.kernel,
    out_type=jax.ShapeDtypeStruct((NC * NP, H), jnp.float32),
    mesh=_sc_mesh,
    scratch_types=[
        pltpu.VMEM((JG, C), jnp.int32),
        pltpu.VMEM((JG, C), jnp.int32),
        pltpu.VMEM((C, H), jnp.float32),
        pltpu.VMEM((C, H), jnp.float32),
        pltpu.VMEM((C, H), jnp.float32),
        pltpu.VMEM_SHARED((NP, H), jnp.float32),
        pltpu.SemaphoreType.DMA,
        pltpu.SemaphoreType.DMA,
        pltpu.SemaphoreType.DMA,
    ],
)
def _sc_agg(h_hbm, srcw, dstw, zeros_hbm, out_hbm, src_v, dst_v, g0, g1, g2,
            acc, sem0, sem1, sem2):
    c = lax.axis_index("c")
    s = lax.axis_index("s")
    wid = c * NS + s
    row0 = s * RPS
    pltpu.sync_copy(zeros_hbm, g0)
    for k in range(RPS // C):
        pltpu.sync_copy(g0, acc.at[pl.ds(row0 + k * C, C)])
    plsc.subcore_barrier()

    bufs = (g0, g1, g2)
    sems = (sem0, sem1, sem2)
    ND = 3
    T = (JG - ND) // ND

    def group(g, carry):
        pltpu.sync_copy(srcw.at[g, wid], src_v)
        pltpu.sync_copy(dstw.at[g, wid], dst_v)
        for k in range(ND):
            pltpu.async_copy(h_hbm.at[src_v.at[k]], bufs[k], sems[k])

        def body(t, carry2):
            j0 = ND * t
            for k in range(ND):
                j = j0 + k
                pltpu.make_async_copy(h_hbm.at[src_v.at[j]], bufs[k], sems[k]).wait()
                pltpu.sync_copy(bufs[k], acc.at[dst_v.at[j]], add=True)
                pltpu.async_copy(h_hbm.at[src_v.at[j + ND]], bufs[k], sems[k])
            return carry2

        lax.fori_loop(0, T, body, 0)
        for j in range(ND * T, JG):
            k = j % ND
            pltpu.make_async_copy(h_hbm.at[src_v.at[j]], bufs[k], sems[k]).wait()
            pltpu.sync_copy(bufs[k], acc.at[dst_v.at[j]], add=True)
            if j + ND < JG:
                pltpu.async_copy(h_hbm.at[src_v.at[j + ND]], bufs[k], sems[k])
        return carry

    lax.fori_loop(0, G, group, 0)
    plsc.subcore_barrier()
    pltpu.sync_copy(acc.at[pl.ds(row0, RPS)],
                    out_hbm.at[pl.ds(c * NP + row0, RPS)])


@functools.partial(
    pl.kernel,
    out_type=jax.ShapeDtypeStruct((NC * NP, H), jnp.float32),
    mesh=_sc_mesh,
    scratch_types=[
        pltpu.VMEM((J, C), jnp.int32),
        pltpu.VMEM((C, H), jnp.float32),
        pltpu.VMEM_SHARED((NP, H), jnp.float32),
    ],
)
def _sc_deg(dstw, zeros_hbm, ones_hbm, out_hbm, dst_v, obuf, acc):
    c = lax.axis_index("c")
    s = lax.axis_index("s")
    wid = c * NS + s
    pltpu.sync_copy(dstw.at[wid], dst_v)
    row0 = s * RPS
    pltpu.sync_copy(zeros_hbm, obuf)
    for k in range(RPS // C):
        pltpu.sync_copy(obuf, acc.at[pl.ds(row0 + k * C, C)])
    pltpu.sync_copy(ones_hbm, obuf)
    plsc.subcore_barrier()

    def body(j, carry):
        pltpu.sync_copy(obuf, acc.at[dst_v.at[j]], add=True)
        return carry

    lax.fori_loop(0, J, body, 0)
    plsc.subcore_barrier()
    pltpu.sync_copy(acc.at[pl.ds(row0, RPS)],
                    out_hbm.at[pl.ds(c * NP + row0, RPS)])


BT = 2048
GT = NP // BT


def _enc_body(x_ref, w1_ref, b1_ref, w2_ref, b2_ref, h_ref, r2_ref):
    i = pl.program_id(0)
    x = x_ref[...]
    r = jnp.maximum(
        jnp.dot(x, w1_ref[...], preferred_element_type=jnp.float32) + b1_ref[...],
        0.0)
    h_ref[...] = (
        jnp.dot(r, w2_ref[...], preferred_element_type=jnp.float32) + b2_ref[...])
    col = lax.broadcasted_iota(jnp.int32, (BT, H), 1)
    m2 = jnp.max(jnp.sum(jnp.where(col < 3, x * x, 0.0), axis=1)).reshape(1, 1)

    @pl.when(i == 0)
    def _():
        r2_ref[...] = m2

    @pl.when(i != 0)
    def _():
        r2_ref[...] = jnp.maximum(r2_ref[...], m2)


_W = pl.BlockSpec((H, H), lambda i: (0, 0))
_B = pl.BlockSpec((1, H), lambda i: (0, 0))
_ROWS = pl.BlockSpec((BT, H), lambda i: (i, 0))

_enc = pl.pallas_call(
    _enc_body,
    grid=(GT,),
    in_specs=[_ROWS, _W, _B, _W, _B],
    out_specs=[_ROWS, pl.BlockSpec((1, 1), lambda i: (0, 0))],
    out_shape=[jax.ShapeDtypeStruct((NP, H), jnp.float32),
               jax.ShapeDtypeStruct((1, 1), jnp.float32)],
)


def _blk_body(p0_ref, p1_ref, d0_ref, d1_ref, h_ref, wl_ref, bl_ref, wr_ref,
              g_ref, be_ref, o_ref):
    deg = d0_ref[:, 0:1] + d1_ref[:, 0:1]
    agg = (p0_ref[...] + p1_ref[...]) / jnp.maximum(deg, 1.0)
    h = h_ref[...]
    z = (jnp.dot(agg, wl_ref[...], preferred_element_type=jnp.float32)
         + bl_ref[...]
         + jnp.dot(h, wr_ref[...], preferred_element_type=jnp.float32))
    mu = jnp.mean(z, axis=1, keepdims=True)
    zc = z - mu
    var = jnp.mean(zc * zc, axis=1, keepdims=True)
    ln = zc / jnp.sqrt(var + 1e-5) * g_ref[...] + be_ref[...]
    o_ref[...] = h + jnp.maximum(ln, 0.0)


_blk = pl.pallas_call(
    _blk_body,
    grid=(GT,),
    in_specs=[_ROWS,
              pl.BlockSpec((BT, H), lambda i: (i + GT, 0)),
              _ROWS,
              pl.BlockSpec((BT, H), lambda i: (i + GT, 0)),
              _ROWS, _W, _B, _W, _B, _B],
    out_specs=_ROWS,
    out_shape=jax.ShapeDtypeStruct((NP, H), jnp.float32),
)


def _blk_heads_body(p0_ref, p1_ref, d0_ref, d1_ref, h_ref, wl_ref, bl_ref,
                    wr_ref, g_ref, be_ref, r2_ref, dw1, db1, dw2, db2, dw3,
                    db3, sw1, sb1, sw2, sb2, sw3, sb3, o_ref):
    deg = d0_ref[:, 0:1] + d1_ref[:, 0:1]
    agg = (p0_ref[...] + p1_ref[...]) / jnp.maximum(deg, 1.0)
    hp = h_ref[...]
    z = (jnp.dot(agg, wl_ref[...], preferred_element_type=jnp.float32)
         + bl_ref[...]
         + jnp.dot(hp, wr_ref[...], preferred_element_type=jnp.float32))
    mu = jnp.mean(z, axis=1, keepdims=True)
    zc = z - mu
    var = jnp.mean(zc * zc, axis=1, keepdims=True)
    ln = zc / jnp.sqrt(var + 1e-5) * g_ref[...] + be_ref[...]
    h = hp + jnp.maximum(ln, 0.0)
    ratio = jnp.sqrt(r2_ref[...])
    d = jnp.maximum(
        jnp.dot(h, dw1[...], preferred_element_type=jnp.float32) + db1[...], 0.0)
    d = jnp.maximum(
        jnp.dot(d, dw2[...], preferred_element_type=jnp.float32) + db2[...], 0.0)
    d = jnp.dot(d, dw3[...], preferred_element_type=jnp.float32) + db3[...]
    s = jnp.maximum(
        jnp.dot(h, sw1[...], preferred_element_type=jnp.float32) + sb1[...], 0.0)
    s = jnp.maximum(
        jnp.dot(s, sw2[...], preferred_element_type=jnp.float32) + sb2[...], 0.0)
    s = jnp.dot(s, sw3[...], preferred_element_type=jnp.float32) + sb3[...]
    o_ref[...] = jnp.concatenate([d, s], axis=1) * ratio


_H2 = H // 2
_blk_heads = pl.pallas_call(
    _blk_heads_body,
    grid=(GT,),
    in_specs=[_ROWS,
              pl.BlockSpec((BT, H), lambda i: (i + GT, 0)),
              _ROWS,
              pl.BlockSpec((BT, H), lambda i: (i + GT, 0)),
              _ROWS, _W, _B, _W, _B, _B,
              pl.BlockSpec((1, 1), lambda i: (0, 0)),
              _W, _B,
              pl.BlockSpec((H, _H2), lambda i: (0, 0)),
              pl.BlockSpec((1, _H2), lambda i: (0, 0)),
              pl.BlockSpec((_H2, 3), lambda i: (0, 0)),
              pl.BlockSpec((1, 3), lambda i: (0, 0)),
              _W, _B,
              pl.BlockSpec((H, _H2), lambda i: (0, 0)),
              pl.BlockSpec((1, _H2), lambda i: (0, 0)),
              pl.BlockSpec((_H2, 1), lambda i: (0, 0)),
              pl.BlockSpec((1, 1), lambda i: (0, 0))],
    out_specs=pl.BlockSpec((BT, 4), lambda i: (i, 0)),
    out_shape=jax.ShapeDtypeStruct((NP, 4), jnp.float32),
)


def kernel(x, edge_index, params):
    src = edge_index[0].astype(jnp.int32).reshape(NW, J, C)
    dst = edge_index[1].astype(jnp.int32).reshape(NW, J, C)
    src_g = src.reshape(NW, G, JG, C).transpose(1, 0, 2, 3)
    dst_g = dst.reshape(NW, G, JG, C).transpose(1, 0, 2, 3)
    zeros_ch = jnp.zeros((C, H), jnp.float32)
    ones_ch = jnp.ones((C, H), jnp.float32)

    degp = _sc_deg(dst, zeros_ch, ones_ch)

    b = lambda v: v.reshape(1, -1)
    x_p = jnp.pad(x, ((0, NP - N), (0, 0)))
    h, r2 = _enc(x_p, params['enc_W1'], b(params['enc_b1']),
                 params['enc_W2'], b(params['enc_b2']))

    for blk in params['blocks'][:-1]:
        parts = _sc_agg(h, src_g, dst_g, zeros_ch)
        h = _blk(parts, parts, degp, degp, h,
                 blk['Wl'], b(blk['bl']), blk['Wr'],
                 b(blk['gamma']), b(blk['beta']))

    blk = params['blocks'][-1]
    parts = _sc_agg(h, src_g, dst_g, zeros_ch)
    out = _blk_heads(parts, parts, degp, degp, h,
                     blk['Wl'], b(blk['bl']), blk['Wr'],
                     b(blk['gamma']), b(blk['beta']), r2,
                     params['d_W1'], b(params['d_b1']),
                     params['d_W2'], b(params['d_b2']),
                     params['d_W3'], b(params['d_b3']),
                     params['s_W1'], b(params['s_b1']),
                     params['s_W2'], b(params['s_b2']),
                     params['s_W3'], b(params['s_b3']))
    return out[:N]
